# baseline (speedup 1.0000x reference)
import functools

import jax
import jax.numpy as jnp
from jax import lax
from jax.experimental import pallas as pl
from jax.experimental.pallas import tpu as pltpu
from jax.experimental.pallas import tpu_sc as plsc

N = 10000
E = 320000
D = 128
NPAD = 10240
NTILES = 32
K = 80
NCHUNK = 128
EPT = NCHUNK * K
EPADDED = NTILES * EPT
NBODY = NCHUNK // 2
RPT = NPAD // 16
ZCOPIES = RPT // K


_MBLK = 400


def _wh_body(f_ref, wt_ref, a1_ref, a2_ref, wh_ref, s_ref, t_ref):
    wh = jnp.dot(f_ref[...], wt_ref[...], preferred_element_type=jnp.float32)
    wh_ref[...] = wh
    s_ref[...] = jnp.dot(wh, a1_ref[...].T, preferred_element_type=jnp.float32)
    t_ref[...] = jnp.dot(wh, a2_ref[...].T, preferred_element_type=jnp.float32)


def _wh_call(feature, wt, a1, a2):
    return pl.pallas_call(
        _wh_body,
        grid=(N // _MBLK,),
        in_specs=[
            pl.BlockSpec((_MBLK, D), lambda i: (i, 0)),
            pl.BlockSpec((D, D), lambda i: (0, 0)),
            pl.BlockSpec((1, D), lambda i: (0, 0)),
            pl.BlockSpec((1, D), lambda i: (0, 0)),
        ],
        out_specs=[
            pl.BlockSpec((_MBLK, D), lambda i: (i, 0)),
            pl.BlockSpec((_MBLK, 1), lambda i: (i, 0)),
            pl.BlockSpec((_MBLK, 1), lambda i: (i, 0)),
        ],
        out_shape=[
            jax.ShapeDtypeStruct((N, D), jnp.float32),
            jax.ShapeDtypeStruct((N, 1), jnp.float32),
            jax.ShapeDtypeStruct((N, 1), jnp.float32),
        ],
    )(feature, wt, a1, a2)


_sc_mesh = plsc.VectorSubcoreMesh(core_axis_name="c", subcore_axis_name="s")


@functools.partial(
    pl.kernel,
    out_type=[
        jax.ShapeDtypeStruct((2, NPAD, D), jnp.float32),
        jax.ShapeDtypeStruct((2, NPAD), jnp.float32),
    ],
    mesh=_sc_mesh,
    compiler_params=pltpu.CompilerParams(needs_layout_passes=False),
    scratch_types=[
        pltpu.VMEM_SHARED((NPAD, D), jnp.float32),
        pltpu.VMEM_SHARED((NPAD,), jnp.float32),
        pltpu.VMEM((N,), jnp.float32),
        pltpu.VMEM((NPAD,), jnp.float32),
        pltpu.VMEM((K,), jnp.int32),
        pltpu.VMEM((K,), jnp.int32),
        pltpu.VMEM((K,), jnp.int32),
        pltpu.VMEM((K,), jnp.int32),
        pltpu.VMEM((K,), jnp.float32),
        pltpu.VMEM((K,), jnp.float32),
        pltpu.VMEM((K, D), jnp.float32),
        pltpu.VMEM((K, D), jnp.float32),
        pltpu.VMEM((RPT,), jnp.float32),
        pltpu.SemaphoreType.DMA,
        pltpu.SemaphoreType.DMA,
        pltpu.SemaphoreType.DMA,
        pltpu.SemaphoreType.DMA,
        pltpu.SemaphoreType.DMA,
    ],
)
def _sc_edges(wh_hbm, s_hbm, t_hbm, src_hbm, dst_hbm,
              numer_out, denom_out,
              numer_sh, denom_sh, s_v, t_v,
              srcA, dstA, srcB, dstB, paA, paB, bufa, bufb, z1d,
              gsemA, gsemB, fsemA, isemA, isemB):
    cid = lax.axis_index("c")
    sid = lax.axis_index("s")
    wid = cid * 16 + sid
    row0 = sid * RPT
    zv = jnp.zeros((16,), jnp.float32)

    def fetch_idx(c, srcX, dstX, isem):
        pltpu.async_copy(src_hbm.at[wid, c], srcX, isem)
        pltpu.async_copy(dst_hbm.at[wid, c], dstX, isem)

    def wait_idx(c, srcX, dstX, isem):
        pltpu.make_async_copy(src_hbm.at[wid, c], srcX, isem).wait()
        pltpu.make_async_copy(dst_hbm.at[wid, c], dstX, isem).wait()

    def weights(srcX, dstX, paX):
        def _w(j, _):
            si = srcX[pl.ds(j * 16, 16)]
            di = dstX[pl.ds(j * 16, 16)]
            e = plsc.load_gather(s_v, [si]) + plsc.load_gather(t_v, [di])
            e = jnp.where(e >= 0.0, e, 0.2 * e)
            paX[pl.ds(j * 16, 16)] = jnp.exp(e)
            return _

        lax.fori_loop(0, K // 16, _w, None)

    def scale(buf, paX):
        def _body(i, _):
            pvec = paX[pl.ds(i * 16, 16)]
            for r in range(16):
                p = pvec[r]
                row = i * 16 + r
                for j in range(D // 16):
                    buf[row, pl.ds(j * 16, 16)] = (
                        buf[row, pl.ds(j * 16, 16)] * p)
            return _

        lax.fori_loop(0, K // 16, _body, None)

    def flush_sync(buf, dstX, paX):
        pltpu.sync_copy(buf, numer_sh.at[dstX], add=True)
        pltpu.sync_copy(paX, denom_sh.at[dstX], add=True)

    pltpu.sync_copy(s_hbm, s_v)
    pltpu.sync_copy(t_hbm, t_v)

    def _zero_rows(i, _):
        for j in range(D // 16):
            bufa[i, pl.ds(j * 16, 16)] = zv
        return _

    lax.fori_loop(0, K, _zero_rows, None)

    def _zero_z1(i, _):
        z1d[pl.ds(i * 16, 16)] = zv
        return _

    lax.fori_loop(0, RPT // 16, _zero_z1, None)

    def _fill_numer(c, _):
        pltpu.sync_copy(bufa, numer_sh.at[pl.ds(row0 + c * K, K)])
        return _

    lax.fori_loop(0, ZCOPIES, _fill_numer, None)
    pltpu.sync_copy(z1d, denom_sh.at[pl.ds(row0, RPT)])

    fetch_idx(jnp.int32(0), srcA, dstA, isemA)
    fetch_idx(jnp.int32(1), srcB, dstB, isemB)

    plsc.subcore_barrier()

    def _pipe(i, _):
        c0 = 2 * i
        c1 = c0 + 1
        wait_idx(c0, srcA, dstA, isemA)
        wait_idx(c1, srcB, dstB, isemB)
        dA = pltpu.async_copy(wh_hbm.at[srcA], bufa, gsemA)
        dB = pltpu.async_copy(wh_hbm.at[srcB], bufb, gsemB)
        weights(srcA, dstA, paA)
        weights(srcB, dstB, paB)
        dA.wait()
        scale(bufa, paA)
        flush_sync(bufa, dstA, paA)
        dB.wait()
        scale(bufb, paB)
        cn0 = jnp.minimum(c0 + 2, NCHUNK - 2)
        cn1 = jnp.minimum(c1 + 2, NCHUNK - 1)
        fetch_idx(cn0, srcA, dstA, isemA)
        flush_sync(bufb, dstB, paB)
        fetch_idx(cn1, srcB, dstB, isemB)
        return _

    lax.fori_loop(0, NBODY, _pipe, None)

    wait_idx(jnp.int32(NCHUNK - 2), srcA, dstA, isemA)
    wait_idx(jnp.int32(NCHUNK - 1), srcB, dstB, isemB)

    plsc.subcore_barrier()

    def _dump(c, _):
        r = row0 + c * K
        pltpu.sync_copy(numer_sh.at[pl.ds(r, K)], bufa)
        pltpu.sync_copy(bufa, numer_out.at[cid, pl.ds(r, K)])
        return _

    lax.fori_loop(0, ZCOPIES, _dump, None)
    pltpu.sync_copy(denom_sh.at[pl.ds(row0, RPT)], z1d)
    pltpu.sync_copy(z1d, denom_out.at[cid, pl.ds(row0, RPT)])


def _final_body(n_ref, d_ref, b_ref, o_ref):
    n = n_ref[...]
    d = d_ref[...]
    num = n[0] + n[1]
    den = d[0] + d[1]
    h = jnp.where(den > 0.0, num / jnp.where(den > 0.0, den, 1.0), 0.0)
    x = h + b_ref[...]
    o_ref[...] = jnp.where(x > 0.0, x, jnp.exp(jnp.minimum(x, 0.0)) - 1.0)


def _final_call(numer, denom3, bias):
    return pl.pallas_call(
        _final_body,
        grid=(N // _MBLK,),
        in_specs=[
            pl.BlockSpec((2, _MBLK, D), lambda i: (0, i, 0)),
            pl.BlockSpec((2, _MBLK, 1), lambda i: (0, i, 0)),
            pl.BlockSpec((1, D), lambda i: (0, 0)),
        ],
        out_specs=pl.BlockSpec((_MBLK, D), lambda i: (i, 0)),
        out_shape=jax.ShapeDtypeStruct((N, D), jnp.float32),
    )(numer, denom3, bias)


def kernel(feature, edge_index, W, a, bias):
    wt = W.T
    a1 = a[:, :D]
    a2 = a[:, D:]
    wh, s2, t2 = _wh_call(feature, wt, a1, a2)
    s = s2.reshape(N)
    t = jnp.concatenate([t2.reshape(N), jnp.zeros((NPAD - N,), jnp.float32)])

    npad_e = EPADDED - E
    src = jnp.concatenate([edge_index[0], jnp.zeros((npad_e,), jnp.int32)])
    dst = jnp.concatenate(
        [edge_index[1],
         N + (jnp.arange(npad_e, dtype=jnp.int32) % (NPAD - N))])
    src3 = src.reshape(NTILES, NCHUNK, K)
    dst3 = dst.reshape(NTILES, NCHUNK, K)

    numer, denom = _sc_edges(wh, s, t, src3, dst3)
    return _final_call(numer, denom.reshape(2, NPAD, 1), bias)


# speedup vs baseline: 1.9739x; 1.9739x over previous
import functools

import jax
import jax.numpy as jnp
from jax import lax
from jax.experimental import pallas as pl
from jax.experimental.pallas import tpu as pltpu
from jax.experimental.pallas import tpu_sc as plsc

N = 10000
E = 320000
D = 128
NPAD = 10240
NTILES = 32
K = 80
NCHUNK = 128
EPT = NCHUNK * K
EPADDED = NTILES * EPT
NBODY = NCHUNK // 2
RPT = NPAD // 16
ZCOPIES = RPT // K


_MBLK = 400


def _wh_body(f_ref, wt_ref, a1_ref, a2_ref, wh_ref, s_ref, t_ref):
    wh = jnp.dot(f_ref[...], wt_ref[...], preferred_element_type=jnp.float32)
    wh_ref[...] = wh
    s_ref[...] = jnp.dot(wh, a1_ref[...].T, preferred_element_type=jnp.float32)
    t_ref[...] = jnp.dot(wh, a2_ref[...].T, preferred_element_type=jnp.float32)


def _wh_call(feature, wt, a1, a2):
    return pl.pallas_call(
        _wh_body,
        grid=(N // _MBLK,),
        in_specs=[
            pl.BlockSpec((_MBLK, D), lambda i: (i, 0)),
            pl.BlockSpec((D, D), lambda i: (0, 0)),
            pl.BlockSpec((1, D), lambda i: (0, 0)),
            pl.BlockSpec((1, D), lambda i: (0, 0)),
        ],
        out_specs=[
            pl.BlockSpec((_MBLK, D), lambda i: (i, 0)),
            pl.BlockSpec((_MBLK, 1), lambda i: (i, 0)),
            pl.BlockSpec((_MBLK, 1), lambda i: (i, 0)),
        ],
        out_shape=[
            jax.ShapeDtypeStruct((N, D), jnp.float32),
            jax.ShapeDtypeStruct((N, 1), jnp.float32),
            jax.ShapeDtypeStruct((N, 1), jnp.float32),
        ],
    )(feature, wt, a1, a2)


_sc_mesh = plsc.VectorSubcoreMesh(core_axis_name="c", subcore_axis_name="s")


@functools.partial(
    pl.kernel,
    out_type=[
        jax.ShapeDtypeStruct((2, NPAD, D), jnp.float32),
        jax.ShapeDtypeStruct((2, NPAD), jnp.float32),
    ],
    mesh=_sc_mesh,
    compiler_params=pltpu.CompilerParams(needs_layout_passes=False),
    scratch_types=[
        pltpu.VMEM_SHARED((NPAD, D), jnp.float32),
        pltpu.VMEM_SHARED((NPAD,), jnp.float32),
        pltpu.VMEM((N,), jnp.float32),
        pltpu.VMEM((NPAD,), jnp.float32),
        pltpu.VMEM((K,), jnp.int32),
        pltpu.VMEM((K,), jnp.int32),
        pltpu.VMEM((K,), jnp.int32),
        pltpu.VMEM((K,), jnp.int32),
        pltpu.VMEM((K,), jnp.float32),
        pltpu.VMEM((K,), jnp.float32),
        pltpu.VMEM((K, D), jnp.float32),
        pltpu.VMEM((K, D), jnp.float32),
        pltpu.VMEM((RPT,), jnp.float32),
        pltpu.SemaphoreType.DMA,
        pltpu.SemaphoreType.DMA,
        pltpu.SemaphoreType.DMA,
        pltpu.SemaphoreType.DMA,
        pltpu.SemaphoreType.DMA,
    ],
)
def _sc_edges(wh_hbm, s_hbm, t_hbm, src_hbm, dst_hbm,
              numer_out, denom_out,
              numer_sh, denom_sh, s_v, t_v,
              srcA, dstA, srcB, dstB, paA, paB, bufa, bufb, z1d,
              gsemA, gsemB, fsemA, isemA, isemB):
    cid = lax.axis_index("c")
    sid = lax.axis_index("s")
    wid = cid * 16 + sid
    row0 = sid * RPT
    zv = jnp.zeros((16,), jnp.float32)

    def fetch_idx(c, srcX, dstX, isem):
        pltpu.async_copy(src_hbm.at[wid, c], srcX, isem)
        pltpu.async_copy(dst_hbm.at[wid, c], dstX, isem)

    def wait_idx(c, srcX, dstX, isem):
        pltpu.make_async_copy(src_hbm.at[wid, c], srcX, isem).wait()
        pltpu.make_async_copy(dst_hbm.at[wid, c], dstX, isem).wait()

    def weights(srcX, dstX, paX):
        def _w(j, _):
            si = srcX[pl.ds(j * 16, 16)]
            di = dstX[pl.ds(j * 16, 16)]
            e = plsc.load_gather(s_v, [si]) + plsc.load_gather(t_v, [di])
            e = jnp.where(e >= 0.0, e, 0.2 * e)
            paX[pl.ds(j * 16, 16)] = jnp.exp(e)
            return _

        lax.fori_loop(0, K // 16, _w, None)

    def scale(buf, paX):
        def _body(i, _):
            pvec = paX[pl.ds(i * 16, 16)]
            for r in range(16):
                p = pvec[r]
                row = i * 16 + r
                for j in range(D // 16):
                    buf[row, pl.ds(j * 16, 16)] = (
                        buf[row, pl.ds(j * 16, 16)] * p)
            return _

        lax.fori_loop(0, K // 16, _body, None)

    def flush_sync(buf, dstX, paX):
        pltpu.sync_copy(buf, numer_sh.at[dstX], add=True)
        pltpu.sync_copy(paX, denom_sh.at[dstX], add=True)

    pltpu.sync_copy(s_hbm, s_v)
    pltpu.sync_copy(t_hbm, t_v)

    def _zero_rows(i, _):
        for j in range(D // 16):
            bufa[i, pl.ds(j * 16, 16)] = zv
        return _

    lax.fori_loop(0, K, _zero_rows, None)

    def _zero_z1(i, _):
        z1d[pl.ds(i * 16, 16)] = zv
        return _

    lax.fori_loop(0, RPT // 16, _zero_z1, None)

    def _fill_numer(c, _):
        pltpu.sync_copy(bufa, numer_sh.at[pl.ds(row0 + c * K, K)])
        return _

    lax.fori_loop(0, ZCOPIES, _fill_numer, None)
    pltpu.sync_copy(z1d, denom_sh.at[pl.ds(row0, RPT)])

    fetch_idx(jnp.int32(0), srcA, dstA, isemA)
    fetch_idx(jnp.int32(1), srcB, dstB, isemB)

    plsc.subcore_barrier()

    def _pipe(i, _):
        c0 = 2 * i
        c1 = c0 + 1
        wait_idx(c0, srcA, dstA, isemA)
        wait_idx(c1, srcB, dstB, isemB)
        dA = pltpu.async_copy(wh_hbm.at[srcA], bufa, gsemA)
        dB = pltpu.async_copy(wh_hbm.at[srcB], bufb, gsemB)
        weights(srcA, dstA, paA)
        weights(srcB, dstB, paB)
        dA.wait()
        scale(bufa, paA)
        flush_sync(bufa, dstA, paA)
        dB.wait()
        scale(bufb, paB)
        cn0 = jnp.minimum(c0 + 2, NCHUNK - 2)
        cn1 = jnp.minimum(c1 + 2, NCHUNK - 1)
        fetch_idx(cn0, srcA, dstA, isemA)
        flush_sync(bufb, dstB, paB)
        fetch_idx(cn1, srcB, dstB, isemB)
        return _

    lax.fori_loop(0, NBODY, _pipe, None)

    wait_idx(jnp.int32(NCHUNK - 2), srcA, dstA, isemA)
    wait_idx(jnp.int32(NCHUNK - 1), srcB, dstB, isemB)

    plsc.subcore_barrier()

    def _dump(c, _):
        r = row0 + c * K
        pltpu.sync_copy(numer_sh.at[pl.ds(r, K)], bufa)
        pltpu.sync_copy(bufa, numer_out.at[cid, pl.ds(r, K)])
        return _

    lax.fori_loop(0, ZCOPIES, _dump, None)
    pltpu.sync_copy(denom_sh.at[pl.ds(row0, RPT)], z1d)
    pltpu.sync_copy(z1d, denom_out.at[cid, pl.ds(row0, RPT)])


def _final_body(n_ref, d_ref, b_ref, o_ref):
    n = n_ref[...]
    d = d_ref[...]
    num = n[0] + n[1]
    den = d[0] + d[1]
    h = jnp.where(den > 0.0, num / jnp.where(den > 0.0, den, 1.0), 0.0)
    x = h + b_ref[...]
    o_ref[...] = jnp.where(x > 0.0, x, jnp.exp(jnp.minimum(x, 0.0)) - 1.0)


def _final_call(numer, denom3, bias):
    return pl.pallas_call(
        _final_body,
        grid=(N // _MBLK,),
        in_specs=[
            pl.BlockSpec((2, _MBLK, D), lambda i: (0, i, 0)),
            pl.BlockSpec((2, _MBLK, 1), lambda i: (0, i, 0)),
            pl.BlockSpec((1, D), lambda i: (0, 0)),
        ],
        out_specs=pl.BlockSpec((_MBLK, D), lambda i: (i, 0)),
        out_shape=jax.ShapeDtypeStruct((N, D), jnp.float32),
    )(numer, denom3, bias)


def kernel(feature, edge_index, W, a, bias):
    wt = W.T
    a1 = a[:, :D]
    a2 = a[:, D:]
    wh, s2, t2 = _wh_call(feature, wt, a1, a2)
    s = s2.reshape(N)
    t = jnp.concatenate([t2.reshape(N), jnp.zeros((NPAD - N,), jnp.float32)])

    ppt = EPT - E // NTILES
    pad_src = jnp.broadcast_to(jnp.arange(ppt, dtype=jnp.int32),
                               (NTILES, ppt))
    pad_dst = jnp.broadcast_to(N + jnp.arange(ppt, dtype=jnp.int32),
                               (NTILES, ppt))
    src3 = jnp.concatenate(
        [edge_index[0].reshape(NTILES, E // NTILES), pad_src],
        axis=1).reshape(NTILES, NCHUNK, K)
    dst3 = jnp.concatenate(
        [edge_index[1].reshape(NTILES, E // NTILES), pad_dst],
        axis=1).reshape(NTILES, NCHUNK, K)

    numer, denom = _sc_edges(wh, s, t, src3, dst3)
    return _final_call(numer, denom.reshape(2, NPAD, 1), bias)


# speedup vs baseline: 2.1162x; 1.0721x over previous
import functools

import jax
import jax.numpy as jnp
from jax import lax
from jax.experimental import pallas as pl
from jax.experimental.pallas import tpu as pltpu
from jax.experimental.pallas import tpu_sc as plsc

N = 10000
E = 320000
D = 128
NPAD = 10240
NTILES = 32
K = 80
NCHUNK = 128
EPT = NCHUNK * K
EPADDED = NTILES * EPT
NBODY = NCHUNK // 2
RPT = NPAD // 16
ZCOPIES = RPT // K


_MBLK = 400


def _wh_body(f_ref, wt_ref, a1_ref, a2_ref, wh_ref, s_ref, t_ref):
    wh = jnp.dot(f_ref[...], wt_ref[...], preferred_element_type=jnp.float32)
    wh_ref[...] = wh
    s_ref[...] = jnp.dot(wh, a1_ref[...].T, preferred_element_type=jnp.float32)
    t_ref[...] = jnp.dot(wh, a2_ref[...].T, preferred_element_type=jnp.float32)


def _wh_call(feature, wt, a1, a2):
    return pl.pallas_call(
        _wh_body,
        grid=(N // _MBLK,),
        in_specs=[
            pl.BlockSpec((_MBLK, D), lambda i: (i, 0)),
            pl.BlockSpec((D, D), lambda i: (0, 0)),
            pl.BlockSpec((1, D), lambda i: (0, 0)),
            pl.BlockSpec((1, D), lambda i: (0, 0)),
        ],
        out_specs=[
            pl.BlockSpec((_MBLK, D), lambda i: (i, 0)),
            pl.BlockSpec((_MBLK, 1), lambda i: (i, 0)),
            pl.BlockSpec((_MBLK, 1), lambda i: (i, 0)),
        ],
        out_shape=[
            jax.ShapeDtypeStruct((N, D), jnp.float32),
            jax.ShapeDtypeStruct((N, 1), jnp.float32),
            jax.ShapeDtypeStruct((N, 1), jnp.float32),
        ],
    )(feature, wt, a1, a2)


_sc_mesh = plsc.VectorSubcoreMesh(core_axis_name="c", subcore_axis_name="s")


@functools.partial(
    pl.kernel,
    out_type=[
        jax.ShapeDtypeStruct((2, NPAD, D), jnp.float32),
        jax.ShapeDtypeStruct((2, NPAD), jnp.float32),
    ],
    mesh=_sc_mesh,
    compiler_params=pltpu.CompilerParams(needs_layout_passes=False),
    scratch_types=[
        pltpu.VMEM_SHARED((NPAD, D), jnp.float32),
        pltpu.VMEM_SHARED((NPAD,), jnp.float32),
        pltpu.VMEM((N,), jnp.float32),
        pltpu.VMEM((NPAD,), jnp.float32),
        pltpu.VMEM((K,), jnp.int32),
        pltpu.VMEM((K,), jnp.int32),
        pltpu.VMEM((K,), jnp.int32),
        pltpu.VMEM((K,), jnp.int32),
        pltpu.VMEM((K,), jnp.float32),
        pltpu.VMEM((K,), jnp.float32),
        pltpu.VMEM((K, D), jnp.float32),
        pltpu.VMEM((K, D), jnp.float32),
        pltpu.VMEM((RPT,), jnp.float32),
        pltpu.SemaphoreType.DMA,
        pltpu.SemaphoreType.DMA,
        pltpu.SemaphoreType.DMA,
        pltpu.SemaphoreType.DMA,
        pltpu.SemaphoreType.DMA,
    ],
)
def _sc_edges(wh_hbm, s_hbm, t_hbm, src_hbm, dst_hbm,
              numer_out, denom_out,
              numer_sh, denom_sh, s_v, t_v,
              srcA, dstA, srcB, dstB, paA, paB, bufa, bufb, z1d,
              gsemA, gsemB, fsemA, isemA, isemB):
    cid = lax.axis_index("c")
    sid = lax.axis_index("s")
    wid = cid * 16 + sid
    row0 = sid * RPT
    zv = jnp.zeros((16,), jnp.float32)

    def fetch_idx(c, srcX, dstX, isem):
        pltpu.async_copy(src_hbm.at[wid, c], srcX, isem)
        pltpu.async_copy(dst_hbm.at[wid, c], dstX, isem)

    def wait_idx(c, srcX, dstX, isem):
        pltpu.make_async_copy(src_hbm.at[wid, c], srcX, isem).wait()
        pltpu.make_async_copy(dst_hbm.at[wid, c], dstX, isem).wait()

    def weights(srcX, dstX, paX):
        def _w(j, _):
            si = srcX[pl.ds(j * 16, 16)]
            di = dstX[pl.ds(j * 16, 16)]
            e = plsc.load_gather(s_v, [si]) + plsc.load_gather(t_v, [di])
            e = jnp.where(e >= 0.0, e, 0.2 * e)
            paX[pl.ds(j * 16, 16)] = jnp.exp(e)
            return _

        lax.fori_loop(0, K // 16, _w, None)

    def scale(buf, paX):
        def _body(i, _):
            pvec = paX[pl.ds(i * 16, 16)]
            for r in range(16):
                p = pvec[r]
                row = i * 16 + r
                for j in range(D // 16):
                    buf[row, pl.ds(j * 16, 16)] = (
                        buf[row, pl.ds(j * 16, 16)] * p)
            return _

        lax.fori_loop(0, K // 16, _body, None)

    def flush_sync(buf, dstX, paX):
        pltpu.sync_copy(buf, numer_sh.at[dstX], add=True)
        pltpu.sync_copy(paX, denom_sh.at[dstX], add=True)

    pltpu.sync_copy(s_hbm, s_v)
    pltpu.sync_copy(t_hbm, t_v)

    def _zero_rows(i, _):
        for j in range(D // 16):
            bufa[i, pl.ds(j * 16, 16)] = zv
        return _

    lax.fori_loop(0, K, _zero_rows, None)

    def _zero_z1(i, _):
        z1d[pl.ds(i * 16, 16)] = zv
        return _

    lax.fori_loop(0, RPT // 16, _zero_z1, None)

    def _fill_numer(c, _):
        pltpu.sync_copy(bufa, numer_sh.at[pl.ds(row0 + c * K, K)])
        return _

    lax.fori_loop(0, ZCOPIES, _fill_numer, None)
    pltpu.sync_copy(z1d, denom_sh.at[pl.ds(row0, RPT)])

    fetch_idx(jnp.int32(0), srcA, dstA, isemA)
    fetch_idx(jnp.int32(1), srcB, dstB, isemB)

    plsc.subcore_barrier()

    def _pipe(i, _):
        c0 = 2 * i
        c1 = c0 + 1
        wait_idx(c0, srcA, dstA, isemA)
        wait_idx(c1, srcB, dstB, isemB)
        dA = pltpu.async_copy(wh_hbm.at[srcA], bufa, gsemA)
        dB = pltpu.async_copy(wh_hbm.at[srcB], bufb, gsemB)
        weights(srcA, dstA, paA)
        weights(srcB, dstB, paB)
        dA.wait()
        scale(bufa, paA)
        fA1 = pltpu.async_copy(bufa, numer_sh.at[dstA], fsemA, add=True)
        fA2 = pltpu.async_copy(paA, denom_sh.at[dstA], fsemA, add=True)
        dB.wait()
        scale(bufb, paB)
        fA1.wait()
        fA2.wait()
        cn0 = jnp.minimum(c0 + 2, NCHUNK - 2)
        cn1 = jnp.minimum(c1 + 2, NCHUNK - 1)
        fetch_idx(cn0, srcA, dstA, isemA)
        flush_sync(bufb, dstB, paB)
        fetch_idx(cn1, srcB, dstB, isemB)
        return _

    lax.fori_loop(0, NBODY, _pipe, None)

    wait_idx(jnp.int32(NCHUNK - 2), srcA, dstA, isemA)
    wait_idx(jnp.int32(NCHUNK - 1), srcB, dstB, isemB)

    plsc.subcore_barrier()

    def _dump(c, _):
        r = row0 + c * K
        pltpu.sync_copy(numer_sh.at[pl.ds(r, K)], bufa)
        pltpu.sync_copy(bufa, numer_out.at[cid, pl.ds(r, K)])
        return _

    lax.fori_loop(0, ZCOPIES, _dump, None)
    pltpu.sync_copy(denom_sh.at[pl.ds(row0, RPT)], z1d)
    pltpu.sync_copy(z1d, denom_out.at[cid, pl.ds(row0, RPT)])


def _final_body(n_ref, d_ref, b_ref, o_ref):
    n = n_ref[...]
    d = d_ref[...]
    num = n[0] + n[1]
    den = d[0] + d[1]
    h = jnp.where(den > 0.0, num / jnp.where(den > 0.0, den, 1.0), 0.0)
    x = h + b_ref[...]
    o_ref[...] = jnp.where(x > 0.0, x, jnp.exp(jnp.minimum(x, 0.0)) - 1.0)


def _final_call(numer, denom3, bias):
    return pl.pallas_call(
        _final_body,
        grid=(N // _MBLK,),
        in_specs=[
            pl.BlockSpec((2, _MBLK, D), lambda i: (0, i, 0)),
            pl.BlockSpec((2, _MBLK, 1), lambda i: (0, i, 0)),
            pl.BlockSpec((1, D), lambda i: (0, 0)),
        ],
        out_specs=pl.BlockSpec((_MBLK, D), lambda i: (i, 0)),
        out_shape=jax.ShapeDtypeStruct((N, D), jnp.float32),
    )(numer, denom3, bias)


def kernel(feature, edge_index, W, a, bias):
    wt = W.T
    a1 = a[:, :D]
    a2 = a[:, D:]
    wh, s2, t2 = _wh_call(feature, wt, a1, a2)
    s = s2.reshape(N)
    t = jnp.concatenate([t2.reshape(N), jnp.zeros((NPAD - N,), jnp.float32)])

    ppt = EPT - E // NTILES
    pad_src = jnp.broadcast_to(jnp.arange(ppt, dtype=jnp.int32),
                               (NTILES, ppt))
    pad_dst = jnp.broadcast_to(N + jnp.arange(ppt, dtype=jnp.int32),
                               (NTILES, ppt))
    src3 = jnp.concatenate(
        [edge_index[0].reshape(NTILES, E // NTILES), pad_src],
        axis=1).reshape(NTILES, NCHUNK, K)
    dst3 = jnp.concatenate(
        [edge_index[1].reshape(NTILES, E // NTILES), pad_dst],
        axis=1).reshape(NTILES, NCHUNK, K)

    numer, denom = _sc_edges(wh, s, t, src3, dst3)
    return _final_call(numer, denom.reshape(2, NPAD, 1), bias)
